# baseline (speedup 1.0000x reference)
import jax
import jax.numpy as jnp
from jax.experimental import pallas as pl
from jax.experimental.pallas import tpu as pltpu

NUM_ENVS = 4096
N_IN = 12
NNZ_OUT = 8


def _compute(w):
    w0, w1, w2, w3, w4, w5, w6, w7, w8, w9, w10, w11 = w
    c12 = 0.5
    c13 = 2.0
    t14 = w0 + w1
    t15 = w2 * w3
    t14 = t14 - t15
    t15 = jnp.sin(w4)
    n0 = jnp.cos(w5)
    n1 = t15 * n0
    n2 = t14 + n1
    n3 = w6 * w6
    n4 = w7 * w7
    n3 = n3 + n4
    n3 = jnp.sqrt(n3)
    n3 = n3 + c12
    n4 = n2 / n3
    n5 = -w8
    n5 = n5 * c12
    n6 = w9 + c13
    n7 = w10 * w11
    n8 = jnp.tan(n4)
    n9 = n5 + n6
    n10 = n7 - n8
    n11 = jnp.sin(n9)
    n12 = jnp.cos(n10)
    n13 = n11 * n12
    t14 = n2 + n13
    t15 = n3 * n3
    return [t14, n4, n9, n10, n11, n12, n13, t15]


def _body(x_ref, o_ref):
    na = NUM_ENVS // 128
    x3 = x_ref[...].reshape(na, N_IN, 128)
    lane = jax.lax.broadcasted_iota(jnp.int32, (na, 128), 1)
    w = []
    for j in range(N_IN):
        pos = N_IN * lane + j
        idx = pos % 128
        cid = pos // 128
        acc = jnp.take_along_axis(x3[:, 0, :], idx, axis=1)
        for c in range(1, N_IN):
            g = jnp.take_along_axis(x3[:, c, :], idx, axis=1)
            acc = jnp.where(cid == c, g, acc)
        w.append(acc)
    outs = _compute(w)
    ps = []
    for p in range(NNZ_OUT):
        idx = 16 * p + lane // NNZ_OUT
        mid = lane % NNZ_OUT
        acc = jnp.take_along_axis(outs[0], idx, axis=1)
        for m in range(1, NNZ_OUT):
            g = jnp.take_along_axis(outs[m], idx, axis=1)
            acc = jnp.where(mid == m, g, acc)
        ps.append(acc)
    o2 = jnp.concatenate(ps, axis=1)
    o_ref[...] = o2.reshape(na, NNZ_OUT, 128).reshape(NUM_ENVS * NNZ_OUT // 128, 128)


def kernel(input_batch):
    x = input_batch.reshape(NUM_ENVS * N_IN // 128, 128)
    out = pl.pallas_call(
        _body,
        out_shape=jax.ShapeDtypeStruct((NUM_ENVS * NNZ_OUT // 128, 128), jnp.float32),
    )(x)
    return out.reshape(NUM_ENVS, NNZ_OUT, 1)


# speedup vs baseline: 5.3789x; 5.3789x over previous
import jax
import jax.numpy as jnp
from jax.experimental import pallas as pl
from jax.experimental.pallas import tpu as pltpu

NUM_ENVS = 4096
N_IN = 12
NNZ_OUT = 8


def _compute(w):
    w0, w1, w2, w3, w4, w5, w6, w7, w8, w9, w10, w11 = w
    c12 = 0.5
    c13 = 2.0
    t14 = w0 + w1
    t15 = w2 * w3
    t14 = t14 - t15
    t15 = jnp.sin(w4)
    n0 = jnp.cos(w5)
    n1 = t15 * n0
    n2 = t14 + n1
    n3 = w6 * w6
    n4 = w7 * w7
    n3 = n3 + n4
    n3 = jnp.sqrt(n3)
    n3 = n3 + c12
    n4 = n2 / n3
    n5 = -w8
    n5 = n5 * c12
    n6 = w9 + c13
    n7 = w10 * w11
    n8 = jnp.tan(n4)
    n9 = n5 + n6
    n10 = n7 - n8
    n11 = jnp.sin(n9)
    n12 = jnp.cos(n10)
    n13 = n11 * n12
    t14 = n2 + n13
    t15 = n3 * n3
    return [t14, n4, n9, n10, n11, n12, n13, t15]


def _body(x_ref, o_ref):
    w = [x_ref[j] for j in range(N_IN)]
    outs = _compute(w)
    o_ref[...] = jnp.stack(outs, axis=0)


def kernel(input_batch):
    x = jax.lax.transpose(input_batch, (1, 2, 0)).reshape(N_IN, NUM_ENVS // 128, 128)
    out = pl.pallas_call(
        _body,
        out_shape=jax.ShapeDtypeStruct((NNZ_OUT, NUM_ENVS // 128, 128), jnp.float32),
    )(x)
    return jax.lax.transpose(out.reshape(NNZ_OUT, 1, NUM_ENVS), (2, 0, 1))
